# Initial kernel scaffold; baseline (speedup 1.0000x reference)
import jax
import jax.numpy as jnp
from jax.experimental import pallas as pl


def kernel(pred_logits, pred_boxes, model_input_sizes):
    raise NotImplementedError("write your pallas kernel here")


# speedup vs baseline: 1.0001x; 1.0001x over previous
import jax
import jax.numpy as jnp
from jax.experimental import pallas as pl

NUM_CLASSES = 80
NUM_TOP = 300


def _box_kernel(boxes_ref, scale_ref, out_ref):
    i = pl.program_id(0)
    b = boxes_ref[...]
    cx, cy, w, h = b[..., 0], b[..., 1], b[..., 2], b[..., 3]
    s = scale_ref[i, :][None, :]
    out = jnp.stack([cx - 0.5 * w, cy - 0.5 * h, cx + 0.5 * w, cy + 0.5 * h], axis=-1)
    out_ref[...] = out * s[:, None, :]


def kernel(pred_logits, pred_boxes, model_input_sizes):
    scale = jnp.tile(model_input_sizes, (1, 2)).astype(jnp.float32)
    bbox = pl.pallas_call(
        _box_kernel,
        grid=(pred_boxes.shape[0],),
        in_specs=[
            pl.BlockSpec((1, 20000, 4), lambda i: (i, 0, 0)),
            pl.BlockSpec((32, 4), lambda i: (0, 0)),
        ],
        out_specs=pl.BlockSpec((1, 20000, 4), lambda i: (i, 0, 0)),
        out_shape=jax.ShapeDtypeStruct(pred_boxes.shape, jnp.float32),
    )(pred_boxes, scale)
    scores = jax.nn.sigmoid(pred_logits)
    flat = scores.reshape(scores.shape[0], -1)
    scores_top, index = jax.lax.top_k(flat, NUM_TOP)
    labels = index % NUM_CLASSES
    qidx = index // NUM_CLASSES
    gather_idx = jnp.repeat(qidx[:, :, None], bbox.shape[-1], axis=2)
    boxes = jnp.take_along_axis(bbox, gather_idx, axis=1)
    return labels, boxes, scores_top


# speedup vs baseline: 14.2134x; 14.2120x over previous
import jax
import jax.numpy as jnp
from jax import lax
from jax.experimental import pallas as pl
from jax.experimental.pallas import tpu as pltpu
from jax.experimental.pallas import tpu_sc as plsc

NUM_CLASSES = 80
K = 300
OUTP = 320
B = 32
Q = 20000
N = Q * NUM_CLASSES
Q4 = Q * 4
DEPTH = 256
CAPT = DEPTH * 16
W = 12800
NWIN = N // W
T0 = 3.45
NEG = -3.0e38
IMAX = 2**31 - 1


def _sc_body(logits, boxes, scale, lab_o, sco_o, x0_o, y0_o, x1_o, y1_o,
             win, cv, ci, ccv, cci, cr, boxr, sclv, tmpv,
             olab, osco, ox0, oy0, ox1, oy1):
    row = lax.axis_index("s") * 2 + lax.axis_index("c")
    rowN = row * N

    pltpu.sync_copy(boxes.at[pl.ds(row * Q4, Q4)], boxr)
    pltpu.sync_copy(scale.at[pl.ds(row * 32, 32)], sclv)

    iota = lax.iota(jnp.int32, 16)
    one_v = jnp.ones((16,), jnp.int32)
    zero_v = jnp.zeros((16,), jnp.int32)

    def lane_sum(x):
        cur = x
        for sh in (1, 2, 4, 8):
            tmpv[pl.ds(0, 16)] = cur
            g = plsc.load_gather(tmpv, [iota ^ sh])
            cur = cur + g
        return cur

    def lane_max(x):
        cur = x
        for sh in (1, 2, 4, 8):
            tmpv[pl.ds(0, 16)] = cur
            g = plsc.load_gather(tmpv, [iota ^ sh])
            cur = jnp.maximum(cur, g)
        return cur

    def lane_exclusive_prefix(x):
        cur = x
        for sh in (1, 2, 4, 8):
            tmpv[pl.ds(0, 16)] = cur
            g = plsc.load_gather(tmpv, [jnp.maximum(iota - sh, 0)])
            cur = cur + jnp.where(iota >= sh, g, zero_v)
        return cur - x

    dmax_v = jnp.full((16,), DEPTH - 1, jnp.int32)

    def extract(t):
        tv = jnp.full((16,), t, jnp.float32)

        def wbody(w, posl):
            pltpu.sync_copy(logits.at[pl.ds(rowN + w * W, W)], win)

            def vbody(k, posl):
                v = win[pl.ds(k * 16, 16)]
                mj = v > tv
                d = jnp.minimum(posl, dmax_v)
                tgt = d * 16 + iota
                plsc.store_scatter(cv, [tgt], v, mask=mj)
                idxv = (w * W + k * 16) + iota
                plsc.store_scatter(ci, [tgt], idxv, mask=mj)
                return posl + jnp.where(mj, one_v, zero_v)

            return lax.fori_loop(0, W // 16, vbody, posl)

        return lax.fori_loop(0, NWIN, wbody, zero_v)

    def stats(posl):
        total = lane_sum(posl)[0]
        maxl = lane_max(posl)[0]
        return total, maxl

    posl0 = extract(jnp.float32(T0))
    tot0, max0 = stats(posl0)

    def f2key(f):
        bits = lax.bitcast_convert_type(f, jnp.uint32)
        return jnp.where((bits >> jnp.uint32(31)) == jnp.uint32(0),
                         bits ^ jnp.uint32(0x80000000), ~bits)

    def key2f(kk):
        bits = jnp.where(kk >= jnp.uint32(0x80000000),
                         kk ^ jnp.uint32(0x80000000), ~kk)
        return lax.bitcast_convert_type(bits, jnp.float32)

    def invalid(total, maxl):
        return (total < K) | (total > CAPT) | (maxl > DEPTH)

    t0k = f2key(jnp.float32(T0))
    too_many0 = (tot0 > CAPT) | (max0 > DEPTH)
    lo0 = jnp.where(too_many0, t0k, jnp.uint32(0))
    hi0 = jnp.where(tot0 < K, t0k, jnp.uint32(0xFFFFFFFF))

    def cond(st):
        _, _, _, total, maxl, it = st
        return invalid(total, maxl) & (it < jnp.int32(40))

    def bod(st):
        lo, hi, _, _, _, it = st
        mid = lo + (hi - lo) // jnp.uint32(2)
        posl = extract(key2f(mid))
        total, maxl = stats(posl)
        too_many = (total > CAPT) | (maxl > DEPTH)
        lo2 = jnp.where(too_many, mid, lo)
        hi2 = jnp.where(total < K, mid, hi)
        return (lo2, hi2, posl, total, maxl, it + 1)

    _, _, posl, total, maxl, _ = lax.while_loop(
        cond, bod, (lo0, hi0, posl0, tot0, max0, jnp.int32(0)))

    neg_v = jnp.full((16,), NEG, jnp.float32)
    imax_v = jnp.full((16,), IMAX, jnp.int32)

    def cfill(i, _):
        ccv[pl.ds(i * 16, 16)] = neg_v
        cci[pl.ds(i * 16, 16)] = imax_v
        return 0

    lax.fori_loop(0, CAPT // 16, cfill, 0)

    base_l = lane_exclusive_prefix(posl)
    cap_v = jnp.full((16,), CAPT - 1, jnp.int32)
    maxd = jnp.minimum(maxl, jnp.int32(DEPTH))

    def cbody(d, _):
        vals = cv[pl.ds(d * 16, 16)]
        idxs = ci[pl.ds(d * 16, 16)]
        mk2 = posl > d
        tgt = jnp.minimum(base_l + d, cap_v)
        plsc.store_scatter(ccv, [tgt], vals, mask=mk2)
        plsc.store_scatter(cci, [tgt], idxs, mask=mk2)
        return 0

    lax.fori_loop(0, maxd, cbody, 0)

    cnt = jnp.minimum(total, jnp.int32(CAPT))
    nb = (cnt + 15) // 16

    rots = [(iota + r) % 16 for r in range(16)]

    def rbody(bi, _):
        b16 = bi * 16
        vt = ccv[pl.ds(b16, 16)]
        it_ = cci[pl.ds(b16, 16)]

        def inner(bs, acc):
            s16 = bs * 16
            for r in range(16):
                idxv = s16 + rots[r]
                vsr = plsc.load_gather(ccv, [idxv])
                isr = plsc.load_gather(cci, [idxv])
                c = (vsr > vt) | ((vsr == vt) & (isr < it_))
                acc = acc + jnp.where(c, one_v, zero_v)
            return acc

        acc = lax.fori_loop(0, nb, inner, zero_v)
        cr[pl.ds(b16, 16)] = acc
        return 0

    lax.fori_loop(0, nb, rbody, 0)

    kv = jnp.full((16,), K, jnp.int32)
    swv = sclv[pl.ds(0, 16)]
    shv = sclv[pl.ds(16, 16)]
    rcpC = jnp.float32(1.0 / NUM_CLASSES)

    def obody(b, _):
        vb = ccv[pl.ds(b * 16, 16)]
        ib = cci[pl.ds(b * 16, 16)]
        rb = cr[pl.ds(b * 16, 16)]
        msk = rb < kv
        rbc = jnp.where(msk, rb, kv)
        s = 1.0 / (1.0 + jnp.exp(-vb))
        q = ((ib.astype(jnp.float32) + 0.5) * rcpC).astype(jnp.int32)
        labv = ib - q * NUM_CLASSES
        plsc.store_scatter(osco, [rbc], s, mask=msk)
        plsc.store_scatter(olab, [rbc], labv, mask=msk)
        q4 = jnp.where(msk, q * 4, zero_v)
        cx = plsc.load_gather(boxr, [q4])
        cy = plsc.load_gather(boxr, [q4 + 1])
        wv = plsc.load_gather(boxr, [q4 + 2])
        hv = plsc.load_gather(boxr, [q4 + 3])
        plsc.store_scatter(ox0, [rbc], (cx - 0.5 * wv) * swv, mask=msk)
        plsc.store_scatter(oy0, [rbc], (cy - 0.5 * hv) * shv, mask=msk)
        plsc.store_scatter(ox1, [rbc], (cx + 0.5 * wv) * swv, mask=msk)
        plsc.store_scatter(oy1, [rbc], (cy + 0.5 * hv) * shv, mask=msk)
        return 0

    lax.fori_loop(0, nb, obody, 0)

    pltpu.sync_copy(olab, lab_o.at[pl.ds(row * OUTP, OUTP)])
    pltpu.sync_copy(osco, sco_o.at[pl.ds(row * OUTP, OUTP)])
    pltpu.sync_copy(ox0, x0_o.at[pl.ds(row * OUTP, OUTP)])
    pltpu.sync_copy(oy0, y0_o.at[pl.ds(row * OUTP, OUTP)])
    pltpu.sync_copy(ox1, x1_o.at[pl.ds(row * OUTP, OUTP)])
    pltpu.sync_copy(oy1, y1_o.at[pl.ds(row * OUTP, OUTP)])


_mesh = plsc.VectorSubcoreMesh(core_axis_name="c", subcore_axis_name="s",
                               num_cores=2, num_subcores=16)

_f32 = jnp.float32
_i32 = jnp.int32

_sc_call = pl.kernel(
    _sc_body,
    out_type=(
        jax.ShapeDtypeStruct((B * OUTP,), _i32),
        jax.ShapeDtypeStruct((B * OUTP,), _f32),
        jax.ShapeDtypeStruct((B * OUTP,), _f32),
        jax.ShapeDtypeStruct((B * OUTP,), _f32),
        jax.ShapeDtypeStruct((B * OUTP,), _f32),
        jax.ShapeDtypeStruct((B * OUTP,), _f32),
    ),
    mesh=_mesh,
    compiler_params=pltpu.CompilerParams(needs_layout_passes=False),
    scratch_types=[
        pltpu.VMEM((W,), _f32),
        pltpu.VMEM((CAPT,), _f32),
        pltpu.VMEM((CAPT,), _i32),
        pltpu.VMEM((CAPT,), _f32),
        pltpu.VMEM((CAPT,), _i32),
        pltpu.VMEM((CAPT,), _i32),
        pltpu.VMEM((Q4,), _f32),
        pltpu.VMEM((32,), _f32),
        pltpu.VMEM((16,), _i32),
        pltpu.VMEM((OUTP,), _i32),
        pltpu.VMEM((OUTP,), _f32),
        pltpu.VMEM((OUTP,), _f32),
        pltpu.VMEM((OUTP,), _f32),
        pltpu.VMEM((OUTP,), _f32),
        pltpu.VMEM((OUTP,), _f32),
    ],
)


def kernel(pred_logits, pred_boxes, model_input_sizes):
    logits = pred_logits.reshape(-1)
    boxesf = pred_boxes.reshape(-1)
    msf = model_input_sizes.astype(jnp.float32)
    scale32 = jnp.concatenate(
        [jnp.broadcast_to(msf[:, :1], (B, 16)),
         jnp.broadcast_to(msf[:, 1:2], (B, 16))], axis=1).reshape(-1)
    lab, sco, x0, y0, x1, y1 = _sc_call(logits, boxesf, scale32)
    lab = lab.reshape(B, OUTP)[:, :K]
    sco = sco.reshape(B, OUTP)[:, :K]
    boxes = jnp.stack(
        [x0.reshape(B, OUTP)[:, :K], y0.reshape(B, OUTP)[:, :K],
         x1.reshape(B, OUTP)[:, :K], y1.reshape(B, OUTP)[:, :K]], axis=-1)
    return lab, boxes, sco


# speedup vs baseline: 33.9708x; 2.3901x over previous
import jax
import jax.numpy as jnp
from jax import lax
from jax.experimental import pallas as pl
from jax.experimental.pallas import tpu as pltpu
from jax.experimental.pallas import tpu_sc as plsc

NUM_CLASSES = 80
K = 300
OUTP = 320
B = 32
Q = 20000
N = Q * NUM_CLASSES
Q4 = Q * 4
DEPTH = 256
CAPT = DEPTH * 16
W = 12800
NWIN = N // W
T0 = 3.45
NEG = -3.0e38
IMAX = 2**31 - 1


def _sc_body(logits, boxes, scale, lab_o, sco_o, x0_o, y0_o, x1_o, y1_o,
             win, cv, ci, ccv, cci, cr, boxr, sclv, tmpv,
             olab, osco, ox0, oy0, ox1, oy1):
    row = lax.axis_index("s") * 2 + lax.axis_index("c")
    rowN = row * N

    pltpu.sync_copy(boxes.at[pl.ds(row * Q4, Q4)], boxr)
    pltpu.sync_copy(scale.at[pl.ds(row * 32, 32)], sclv)

    iota = lax.iota(jnp.int32, 16)
    one_v = jnp.ones((16,), jnp.int32)
    zero_v = jnp.zeros((16,), jnp.int32)

    def lane_sum(x):
        cur = x
        for sh in (1, 2, 4, 8):
            tmpv[pl.ds(0, 16)] = cur
            g = plsc.load_gather(tmpv, [iota ^ sh])
            cur = cur + g
        return cur

    def lane_max(x):
        cur = x
        for sh in (1, 2, 4, 8):
            tmpv[pl.ds(0, 16)] = cur
            g = plsc.load_gather(tmpv, [iota ^ sh])
            cur = jnp.maximum(cur, g)
        return cur

    def lane_exclusive_prefix(x):
        cur = x
        for sh in (1, 2, 4, 8):
            tmpv[pl.ds(0, 16)] = cur
            g = plsc.load_gather(tmpv, [jnp.maximum(iota - sh, 0)])
            cur = cur + jnp.where(iota >= sh, g, zero_v)
        return cur - x

    dmax_v = jnp.full((16,), DEPTH - 1, jnp.int32)
    i80 = iota * NUM_CLASSES

    def extract(t):
        tv = jnp.full((16,), t, jnp.float32)

        def wbody(w, st):
            posl, q0, cc, ib = st
            pltpu.sync_copy(logits.at[pl.ds(rowN + w * W, W)], win)

            def vbody(k, st):
                posl, q0, cc, ib = st
                v = win[pl.ds(k * 16, 16)]
                mj = v > tv
                d = jnp.minimum(posl, dmax_v)
                tgt = d * 16 + iota
                plsc.store_scatter(cv, [tgt], v, mask=mj)
                idxv = ib + i80
                plsc.store_scatter(ci, [tgt], idxv, mask=mj)
                posl = posl + jnp.where(mj, one_v, zero_v)
                q0n = q0 + 16
                wrap = q0n == Q
                q0 = jnp.where(wrap, 0, q0n)
                ib = jnp.where(wrap, cc + 1, ib + 16 * NUM_CLASSES)
                cc = cc + wrap.astype(jnp.int32)
                return (posl, q0, cc, ib)

            return lax.fori_loop(0, W // 16, vbody, (posl, q0, cc, ib))

        st = lax.fori_loop(0, NWIN, wbody,
                           (zero_v, jnp.int32(0), jnp.int32(0), jnp.int32(0)))
        return st[0]

    def stats(posl):
        total = lane_sum(posl)[0]
        maxl = lane_max(posl)[0]
        return total, maxl

    posl0 = extract(jnp.float32(T0))
    tot0, max0 = stats(posl0)

    def f2key(f):
        bits = lax.bitcast_convert_type(f, jnp.uint32)
        return jnp.where((bits >> jnp.uint32(31)) == jnp.uint32(0),
                         bits ^ jnp.uint32(0x80000000), ~bits)

    def key2f(kk):
        bits = jnp.where(kk >= jnp.uint32(0x80000000),
                         kk ^ jnp.uint32(0x80000000), ~kk)
        return lax.bitcast_convert_type(bits, jnp.float32)

    def invalid(total, maxl):
        return (total < K) | (total > CAPT) | (maxl > DEPTH)

    t0k = f2key(jnp.float32(T0))
    too_many0 = (tot0 > CAPT) | (max0 > DEPTH)
    lo0 = jnp.where(too_many0, t0k, jnp.uint32(0))
    hi0 = jnp.where(tot0 < K, t0k, jnp.uint32(0xFFFFFFFF))

    def cond(st):
        _, _, _, total, maxl, it = st
        return invalid(total, maxl) & (it < jnp.int32(40))

    def bod(st):
        lo, hi, _, _, _, it = st
        mid = lo + (hi - lo) // jnp.uint32(2)
        posl = extract(key2f(mid))
        total, maxl = stats(posl)
        too_many = (total > CAPT) | (maxl > DEPTH)
        lo2 = jnp.where(too_many, mid, lo)
        hi2 = jnp.where(total < K, mid, hi)
        return (lo2, hi2, posl, total, maxl, it + 1)

    _, _, posl, total, maxl, _ = lax.while_loop(
        cond, bod, (lo0, hi0, posl0, tot0, max0, jnp.int32(0)))

    neg_v = jnp.full((16,), NEG, jnp.float32)
    imax_v = jnp.full((16,), IMAX, jnp.int32)

    def cfill(i, _):
        ccv[pl.ds(i * 16, 16)] = neg_v
        cci[pl.ds(i * 16, 16)] = imax_v
        return 0

    lax.fori_loop(0, CAPT // 16, cfill, 0)

    base_l = lane_exclusive_prefix(posl)
    cap_v = jnp.full((16,), CAPT - 1, jnp.int32)
    maxd = jnp.minimum(maxl, jnp.int32(DEPTH))

    def cbody(d, _):
        vals = cv[pl.ds(d * 16, 16)]
        idxs = ci[pl.ds(d * 16, 16)]
        mk2 = posl > d
        tgt = jnp.minimum(base_l + d, cap_v)
        plsc.store_scatter(ccv, [tgt], vals, mask=mk2)
        plsc.store_scatter(cci, [tgt], idxs, mask=mk2)
        return 0

    lax.fori_loop(0, maxd, cbody, 0)

    cnt = jnp.minimum(total, jnp.int32(CAPT))
    nb = (cnt + 15) // 16

    rots = [(iota + r) % 16 for r in range(16)]

    def rbody(bi, _):
        b16 = bi * 16
        vt = ccv[pl.ds(b16, 16)]
        it_ = cci[pl.ds(b16, 16)]

        def inner(bs, acc):
            s16 = bs * 16
            for r in range(16):
                idxv = s16 + rots[r]
                vsr = plsc.load_gather(ccv, [idxv])
                isr = plsc.load_gather(cci, [idxv])
                c = (vsr > vt) | ((vsr == vt) & (isr < it_))
                acc = acc + jnp.where(c, one_v, zero_v)
            return acc

        acc = lax.fori_loop(0, nb, inner, zero_v)
        cr[pl.ds(b16, 16)] = acc
        return 0

    lax.fori_loop(0, nb, rbody, 0)

    kv = jnp.full((16,), K, jnp.int32)
    swv = sclv[pl.ds(0, 16)]
    shv = sclv[pl.ds(16, 16)]
    rcpC = jnp.float32(1.0 / NUM_CLASSES)

    def obody(b, _):
        vb = ccv[pl.ds(b * 16, 16)]
        ib = cci[pl.ds(b * 16, 16)]
        rb = cr[pl.ds(b * 16, 16)]
        msk = rb < kv
        rbc = jnp.where(msk, rb, kv)
        s = 1.0 / (1.0 + jnp.exp(-vb))
        q = ((ib.astype(jnp.float32) + 0.5) * rcpC).astype(jnp.int32)
        labv = ib - q * NUM_CLASSES
        plsc.store_scatter(osco, [rbc], s, mask=msk)
        plsc.store_scatter(olab, [rbc], labv, mask=msk)
        q_ = jnp.where(msk, q, zero_v)
        cx = plsc.load_gather(boxr, [q_])
        cy = plsc.load_gather(boxr, [q_ + Q])
        wv = plsc.load_gather(boxr, [q_ + 2 * Q])
        hv = plsc.load_gather(boxr, [q_ + 3 * Q])
        plsc.store_scatter(ox0, [rbc], (cx - 0.5 * wv) * swv, mask=msk)
        plsc.store_scatter(oy0, [rbc], (cy - 0.5 * hv) * shv, mask=msk)
        plsc.store_scatter(ox1, [rbc], (cx + 0.5 * wv) * swv, mask=msk)
        plsc.store_scatter(oy1, [rbc], (cy + 0.5 * hv) * shv, mask=msk)
        return 0

    lax.fori_loop(0, nb, obody, 0)

    pltpu.sync_copy(olab, lab_o.at[pl.ds(row * OUTP, OUTP)])
    pltpu.sync_copy(osco, sco_o.at[pl.ds(row * OUTP, OUTP)])
    pltpu.sync_copy(ox0, x0_o.at[pl.ds(row * OUTP, OUTP)])
    pltpu.sync_copy(oy0, y0_o.at[pl.ds(row * OUTP, OUTP)])
    pltpu.sync_copy(ox1, x1_o.at[pl.ds(row * OUTP, OUTP)])
    pltpu.sync_copy(oy1, y1_o.at[pl.ds(row * OUTP, OUTP)])


_mesh = plsc.VectorSubcoreMesh(core_axis_name="c", subcore_axis_name="s",
                               num_cores=2, num_subcores=16)

_f32 = jnp.float32
_i32 = jnp.int32

_sc_call = pl.kernel(
    _sc_body,
    out_type=(
        jax.ShapeDtypeStruct((B * OUTP,), _i32),
        jax.ShapeDtypeStruct((B * OUTP,), _f32),
        jax.ShapeDtypeStruct((B * OUTP,), _f32),
        jax.ShapeDtypeStruct((B * OUTP,), _f32),
        jax.ShapeDtypeStruct((B * OUTP,), _f32),
        jax.ShapeDtypeStruct((B * OUTP,), _f32),
    ),
    mesh=_mesh,
    compiler_params=pltpu.CompilerParams(needs_layout_passes=False),
    scratch_types=[
        pltpu.VMEM((W,), _f32),
        pltpu.VMEM((CAPT,), _f32),
        pltpu.VMEM((CAPT,), _i32),
        pltpu.VMEM((CAPT,), _f32),
        pltpu.VMEM((CAPT,), _i32),
        pltpu.VMEM((CAPT,), _i32),
        pltpu.VMEM((Q4,), _f32),
        pltpu.VMEM((32,), _f32),
        pltpu.VMEM((16,), _i32),
        pltpu.VMEM((OUTP,), _i32),
        pltpu.VMEM((OUTP,), _f32),
        pltpu.VMEM((OUTP,), _f32),
        pltpu.VMEM((OUTP,), _f32),
        pltpu.VMEM((OUTP,), _f32),
        pltpu.VMEM((OUTP,), _f32),
    ],
)


def kernel(pred_logits, pred_boxes, model_input_sizes):
    logits = pred_logits.transpose(0, 2, 1).reshape(-1)
    boxesf = pred_boxes.transpose(0, 2, 1).reshape(-1)
    msf = model_input_sizes.astype(jnp.float32)
    scale32 = jnp.concatenate(
        [jnp.broadcast_to(msf[:, :1], (B, 16)),
         jnp.broadcast_to(msf[:, 1:2], (B, 16))], axis=1).reshape(-1)
    lab, sco, x0, y0, x1, y1 = _sc_call(logits, boxesf, scale32)
    lab = lab.reshape(B, OUTP)[:, :K]
    sco = sco.reshape(B, OUTP)[:, :K]
    boxes = jnp.stack(
        [x0.reshape(B, OUTP)[:, :K], y0.reshape(B, OUTP)[:, :K],
         x1.reshape(B, OUTP)[:, :K], y1.reshape(B, OUTP)[:, :K]], axis=-1)
    return lab, boxes, sco


# speedup vs baseline: 38.2353x; 1.1255x over previous
import jax
import jax.numpy as jnp
from jax import lax
from jax.experimental import pallas as pl
from jax.experimental.pallas import tpu as pltpu
from jax.experimental.pallas import tpu_sc as plsc

NUM_CLASSES = 80
K = 300
OUTP = 320
B = 32
Q = 20000
N = Q * NUM_CLASSES
Q4 = Q * 4
DEPTH = 256
CAPT = DEPTH * 16
W = 4000
NWIN = N // W
UNR = 10
T0 = 3.45
NEG = -3.0e38
IMAX = 2**31 - 1


def _sc_body(logits, boxes, scale, lab_o, sco_o, x0_o, y0_o, x1_o, y1_o,
             win, cv, ci, ccv, cci, cr, boxr, sclv, tmpv,
             olab, osco, ox0, oy0, ox1, oy1, dmasem):
    row = lax.axis_index("s") * 2 + lax.axis_index("c")
    rowN = row * N

    pltpu.sync_copy(boxes.at[pl.ds(row * Q4, Q4)], boxr)
    pltpu.sync_copy(scale.at[pl.ds(row * 32, 32)], sclv)

    iota = lax.iota(jnp.int32, 16)
    one_v = jnp.ones((16,), jnp.int32)
    zero_v = jnp.zeros((16,), jnp.int32)

    def lane_sum(x):
        cur = x
        for sh in (1, 2, 4, 8):
            tmpv[pl.ds(0, 16)] = cur
            g = plsc.load_gather(tmpv, [iota ^ sh])
            cur = cur + g
        return cur

    def lane_max(x):
        cur = x
        for sh in (1, 2, 4, 8):
            tmpv[pl.ds(0, 16)] = cur
            g = plsc.load_gather(tmpv, [iota ^ sh])
            cur = jnp.maximum(cur, g)
        return cur

    def lane_exclusive_prefix(x):
        cur = x
        for sh in (1, 2, 4, 8):
            tmpv[pl.ds(0, 16)] = cur
            g = plsc.load_gather(tmpv, [jnp.maximum(iota - sh, 0)])
            cur = cur + jnp.where(iota >= sh, g, zero_v)
        return cur - x

    dmax32 = jnp.full((16,), (DEPTH - 1) * 32, jnp.int32)
    i80 = iota * NUM_CLASSES
    iotaB = iota + 16
    step32 = jnp.full((16,), 32, jnp.int32)

    def extract(t):
        tv = jnp.full((16,), t, jnp.float32)
        VPW = W // 16

        def issue(w, buf):
            return pltpu.async_copy(
                logits.at[pl.ds(rowN + w * W, W)],
                win.at[pl.ds(buf * W, W)], dmasem)

        issue(0, 0)

        def wbody(w, st):
            pA, pB, q0, cc, ib = st
            cur = lax.rem(w, 2)
            pltpu.make_async_copy(
                logits.at[pl.ds(rowN + w * W, W)],
                win.at[pl.ds(cur * W, W)], dmasem).wait()

            @pl.when(w + 1 < NWIN)
            def _():
                issue(w + 1, 1 - cur)

            wbase = cur * W

            def vbody(k, st):
                pA, pB = st
                kb = jnp.full((16,), ib + k * (UNR * 16 * NUM_CLASSES),
                              jnp.int32) + i80
                for j in range(UNR):
                    v = win[pl.ds(wbase + (k * UNR + j) * 16, 16)]
                    mj = v > tv
                    idxv = kb + (j * 16 * NUM_CLASSES)
                    if j % 2 == 0:
                        d = jnp.minimum(pA, dmax32)
                        tgt = d + iota
                        pA = pA + jnp.where(mj, step32, zero_v)
                    else:
                        d = jnp.minimum(pB, dmax32)
                        tgt = d + iotaB
                        pB = pB + jnp.where(mj, step32, zero_v)
                    plsc.store_scatter(cv, [tgt], v, mask=mj)
                    plsc.store_scatter(ci, [tgt], idxv, mask=mj)
                return (pA, pB)

            pA, pB = lax.fori_loop(0, VPW // UNR, vbody, (pA, pB))
            q0n = q0 + W
            wrap = q0n == Q
            q0 = jnp.where(wrap, 0, q0n)
            ib = jnp.where(wrap, cc + 1, ib + W * NUM_CLASSES)
            cc = cc + wrap.astype(jnp.int32)
            return (pA, pB, q0, cc, ib)

        st = lax.fori_loop(0, NWIN, wbody,
                           (zero_v, zero_v, jnp.int32(0), jnp.int32(0),
                            jnp.int32(0)))
        return st[0] >> 5, st[1] >> 5

    def stats(pA, pB):
        total = lane_sum(pA + pB)[0]
        maxl = lane_max(jnp.maximum(pA, pB))[0]
        return total, maxl

    pA0, pB0 = extract(jnp.float32(T0))
    tot0, max0 = stats(pA0, pB0)

    def f2key(f):
        bits = lax.bitcast_convert_type(f, jnp.uint32)
        return jnp.where((bits >> jnp.uint32(31)) == jnp.uint32(0),
                         bits ^ jnp.uint32(0x80000000), ~bits)

    def key2f(kk):
        bits = jnp.where(kk >= jnp.uint32(0x80000000),
                         kk ^ jnp.uint32(0x80000000), ~kk)
        return lax.bitcast_convert_type(bits, jnp.float32)

    def invalid(total, maxl):
        return (total < K) | (total > CAPT) | (maxl > DEPTH)

    t0k = f2key(jnp.float32(T0))
    too_many0 = (tot0 > CAPT) | (max0 > DEPTH)
    lo0 = jnp.where(too_many0, t0k, jnp.uint32(0))
    hi0 = jnp.where(tot0 < K, t0k, jnp.uint32(0xFFFFFFFF))

    def cond(st):
        _, _, _, _, total, maxl, it = st
        return invalid(total, maxl) & (it < jnp.int32(40))

    def bod(st):
        lo, hi, _, _, _, _, it = st
        mid = lo + (hi - lo) // jnp.uint32(2)
        pA, pB = extract(key2f(mid))
        total, maxl = stats(pA, pB)
        too_many = (total > CAPT) | (maxl > DEPTH)
        lo2 = jnp.where(too_many, mid, lo)
        hi2 = jnp.where(total < K, mid, hi)
        return (lo2, hi2, pA, pB, total, maxl, it + 1)

    _, _, pA, pB, total, maxl, _ = lax.while_loop(
        cond, bod, (lo0, hi0, pA0, pB0, tot0, max0, jnp.int32(0)))

    neg_v = jnp.full((16,), NEG, jnp.float32)
    imax_v = jnp.full((16,), IMAX, jnp.int32)

    def cfill(i, _):
        ccv[pl.ds(i * 16, 16)] = neg_v
        cci[pl.ds(i * 16, 16)] = imax_v
        return 0

    lax.fori_loop(0, CAPT // 16, cfill, 0)

    totA = lane_sum(pA)
    base_a = lane_exclusive_prefix(pA)
    base_b = lane_exclusive_prefix(pB) + totA
    cap_v = jnp.full((16,), CAPT - 1, jnp.int32)
    maxd = jnp.minimum(maxl, jnp.int32(DEPTH))

    def cbody(d, _):
        vals = cv[pl.ds(d * 32, 16)]
        idxs = ci[pl.ds(d * 32, 16)]
        mk2 = pA > d
        tgt = jnp.minimum(base_a + d, cap_v)
        plsc.store_scatter(ccv, [tgt], vals, mask=mk2)
        plsc.store_scatter(cci, [tgt], idxs, mask=mk2)
        valsb = cv[pl.ds(d * 32 + 16, 16)]
        idxsb = ci[pl.ds(d * 32 + 16, 16)]
        mk3 = pB > d
        tgtb = jnp.minimum(base_b + d, cap_v)
        plsc.store_scatter(ccv, [tgtb], valsb, mask=mk3)
        plsc.store_scatter(cci, [tgtb], idxsb, mask=mk3)
        return 0

    lax.fori_loop(0, maxd, cbody, 0)

    cnt = jnp.minimum(total, jnp.int32(CAPT))
    nb = (cnt + 15) // 16

    rots = [(iota + r) % 16 for r in range(16)]

    def rbody(bi, _):
        b16 = bi * 16
        vt = ccv[pl.ds(b16, 16)]
        it_ = cci[pl.ds(b16, 16)]

        def inner(bs, acc):
            s16 = bs * 16
            for r in range(16):
                idxv = s16 + rots[r]
                vsr = plsc.load_gather(ccv, [idxv])
                isr = plsc.load_gather(cci, [idxv])
                c = (vsr > vt) | ((vsr == vt) & (isr < it_))
                acc = acc + jnp.where(c, one_v, zero_v)
            return acc

        acc = lax.fori_loop(0, nb, inner, zero_v)
        cr[pl.ds(b16, 16)] = acc
        return 0

    lax.fori_loop(0, nb, rbody, 0)

    kv = jnp.full((16,), K, jnp.int32)
    swv = sclv[pl.ds(0, 16)]
    shv = sclv[pl.ds(16, 16)]
    rcpC = jnp.float32(1.0 / NUM_CLASSES)

    def obody(b, _):
        vb = ccv[pl.ds(b * 16, 16)]
        ib = cci[pl.ds(b * 16, 16)]
        rb = cr[pl.ds(b * 16, 16)]
        msk = rb < kv
        rbc = jnp.where(msk, rb, kv)
        s = 1.0 / (1.0 + jnp.exp(-vb))
        q = ((ib.astype(jnp.float32) + 0.5) * rcpC).astype(jnp.int32)
        labv = ib - q * NUM_CLASSES
        plsc.store_scatter(osco, [rbc], s, mask=msk)
        plsc.store_scatter(olab, [rbc], labv, mask=msk)
        q_ = jnp.where(msk, q, zero_v)
        cx = plsc.load_gather(boxr, [q_])
        cy = plsc.load_gather(boxr, [q_ + Q])
        wv = plsc.load_gather(boxr, [q_ + 2 * Q])
        hv = plsc.load_gather(boxr, [q_ + 3 * Q])
        plsc.store_scatter(ox0, [rbc], (cx - 0.5 * wv) * swv, mask=msk)
        plsc.store_scatter(oy0, [rbc], (cy - 0.5 * hv) * shv, mask=msk)
        plsc.store_scatter(ox1, [rbc], (cx + 0.5 * wv) * swv, mask=msk)
        plsc.store_scatter(oy1, [rbc], (cy + 0.5 * hv) * shv, mask=msk)
        return 0

    lax.fori_loop(0, nb, obody, 0)

    pltpu.sync_copy(olab, lab_o.at[pl.ds(row * OUTP, OUTP)])
    pltpu.sync_copy(osco, sco_o.at[pl.ds(row * OUTP, OUTP)])
    pltpu.sync_copy(ox0, x0_o.at[pl.ds(row * OUTP, OUTP)])
    pltpu.sync_copy(oy0, y0_o.at[pl.ds(row * OUTP, OUTP)])
    pltpu.sync_copy(ox1, x1_o.at[pl.ds(row * OUTP, OUTP)])
    pltpu.sync_copy(oy1, y1_o.at[pl.ds(row * OUTP, OUTP)])


_mesh = plsc.VectorSubcoreMesh(core_axis_name="c", subcore_axis_name="s",
                               num_cores=2, num_subcores=16)

_f32 = jnp.float32
_i32 = jnp.int32

_sc_call = pl.kernel(
    _sc_body,
    out_type=(
        jax.ShapeDtypeStruct((B * OUTP,), _i32),
        jax.ShapeDtypeStruct((B * OUTP,), _f32),
        jax.ShapeDtypeStruct((B * OUTP,), _f32),
        jax.ShapeDtypeStruct((B * OUTP,), _f32),
        jax.ShapeDtypeStruct((B * OUTP,), _f32),
        jax.ShapeDtypeStruct((B * OUTP,), _f32),
    ),
    mesh=_mesh,
    compiler_params=pltpu.CompilerParams(needs_layout_passes=False),
    scratch_types=[
        pltpu.VMEM((2 * W,), _f32),
        pltpu.VMEM((2 * CAPT,), _f32),
        pltpu.VMEM((2 * CAPT,), _i32),
        pltpu.VMEM((CAPT,), _f32),
        pltpu.VMEM((CAPT,), _i32),
        pltpu.VMEM((CAPT,), _i32),
        pltpu.VMEM((Q4,), _f32),
        pltpu.VMEM((32,), _f32),
        pltpu.VMEM((16,), _i32),
        pltpu.VMEM((OUTP,), _i32),
        pltpu.VMEM((OUTP,), _f32),
        pltpu.VMEM((OUTP,), _f32),
        pltpu.VMEM((OUTP,), _f32),
        pltpu.VMEM((OUTP,), _f32),
        pltpu.VMEM((OUTP,), _f32),
        pltpu.SemaphoreType.DMA,
    ],
)


def kernel(pred_logits, pred_boxes, model_input_sizes):
    logits = pred_logits.transpose(0, 2, 1).reshape(-1)
    boxesf = pred_boxes.transpose(0, 2, 1).reshape(-1)
    msf = model_input_sizes.astype(jnp.float32)
    scale32 = jnp.concatenate(
        [jnp.broadcast_to(msf[:, :1], (B, 16)),
         jnp.broadcast_to(msf[:, 1:2], (B, 16))], axis=1).reshape(-1)
    lab, sco, x0, y0, x1, y1 = _sc_call(logits, boxesf, scale32)
    lab = lab.reshape(B, OUTP)[:, :K]
    sco = sco.reshape(B, OUTP)[:, :K]
    boxes = jnp.stack(
        [x0.reshape(B, OUTP)[:, :K], y0.reshape(B, OUTP)[:, :K],
         x1.reshape(B, OUTP)[:, :K], y1.reshape(B, OUTP)[:, :K]], axis=-1)
    return lab, boxes, sco


# speedup vs baseline: 67.5478x; 1.7666x over previous
import jax
import jax.numpy as jnp
from jax import lax
from jax.experimental import pallas as pl
from jax.experimental.pallas import tpu as pltpu
from jax.experimental.pallas import tpu_sc as plsc

NUM_CLASSES = 80
K = 300
OUTP = 320
B = 32
Q = 20000
N = Q * NUM_CLASSES
Q4 = Q * 4
DEPTH = 256
CAPT = DEPTH * 16
W = 4000
NWIN = N // W
UNR = 10
T0 = 3.45
NEG = -3.0e38
IMAX = 2**31 - 1


def _sc_body(logits, boxes, scale, lab_o, sco_o, x0_o, y0_o, x1_o, y1_o,
             win, cv, ci, ccv, cci, cr, boxr, sclv, tmpv,
             olab, osco, ox0, oy0, ox1, oy1, dmasem):
    row = lax.axis_index("s") * 2 + lax.axis_index("c")
    rowN = row * N

    pltpu.sync_copy(boxes.at[pl.ds(row * Q4, Q4)], boxr)
    pltpu.sync_copy(scale.at[pl.ds(row * 32, 32)], sclv)

    iota = lax.iota(jnp.int32, 16)
    one_v = jnp.ones((16,), jnp.int32)
    zero_v = jnp.zeros((16,), jnp.int32)

    def lane_sum(x):
        cur = x
        for sh in (1, 2, 4, 8):
            tmpv[pl.ds(0, 16)] = cur
            g = plsc.load_gather(tmpv, [iota ^ sh])
            cur = cur + g
        return cur

    def lane_max(x):
        cur = x
        for sh in (1, 2, 4, 8):
            tmpv[pl.ds(0, 16)] = cur
            g = plsc.load_gather(tmpv, [iota ^ sh])
            cur = jnp.maximum(cur, g)
        return cur

    def lane_exclusive_prefix(x):
        cur = x
        for sh in (1, 2, 4, 8):
            tmpv[pl.ds(0, 16)] = cur
            g = plsc.load_gather(tmpv, [jnp.maximum(iota - sh, 0)])
            cur = cur + jnp.where(iota >= sh, g, zero_v)
        return cur - x

    dmax32 = jnp.full((16,), (DEPTH - 1) * 32, jnp.int32)
    i80 = iota * NUM_CLASSES
    iotaB = iota + 16
    step32 = jnp.full((16,), 32, jnp.int32)

    def extract(t):
        tv = jnp.full((16,), t, jnp.float32)
        VPW = W // 16

        def issue(w, buf):
            return pltpu.async_copy(
                logits.at[pl.ds(rowN + w * W, W)],
                win.at[pl.ds(buf * W, W)], dmasem)

        issue(0, 0)

        def wbody(w, st):
            pA, pB, q0, cc, ib = st
            cur = lax.rem(w, 2)
            pltpu.make_async_copy(
                logits.at[pl.ds(rowN + w * W, W)],
                win.at[pl.ds(cur * W, W)], dmasem).wait()

            @pl.when(w + 1 < NWIN)
            def _():
                issue(w + 1, 1 - cur)

            wbase = cur * W

            def vbody(k, st):
                pA, pB = st
                base = wbase + k * (UNR * 16)
                kb = jnp.full((16,), ib + k * (UNR * 16 * NUM_CLASSES),
                              jnp.int32) + i80
                vs = [win[pl.ds(base + j * 16, 16)] for j in range(UNR)]
                ms = [v > tv for v in vs]
                tgts = []
                for j in range(UNR):
                    if j % 2 == 0:
                        tgts.append(jnp.minimum(pA, dmax32) + iota)
                        pA = pA + jnp.where(ms[j], step32, zero_v)
                    else:
                        tgts.append(jnp.minimum(pB, dmax32) + iotaB)
                        pB = pB + jnp.where(ms[j], step32, zero_v)
                for j in range(UNR):
                    plsc.store_scatter(cv, [tgts[j]], vs[j], mask=ms[j])
                    plsc.store_scatter(ci, [tgts[j]],
                                       kb + (j * 16 * NUM_CLASSES),
                                       mask=ms[j])
                return (pA, pB)

            pA, pB = lax.fori_loop(0, VPW // UNR, vbody, (pA, pB))
            q0n = q0 + W
            wrap = q0n == Q
            q0 = jnp.where(wrap, 0, q0n)
            ib = jnp.where(wrap, cc + 1, ib + W * NUM_CLASSES)
            cc = cc + wrap.astype(jnp.int32)
            return (pA, pB, q0, cc, ib)

        st = lax.fori_loop(0, NWIN, wbody,
                           (zero_v, zero_v, jnp.int32(0), jnp.int32(0),
                            jnp.int32(0)))
        return st[0] >> 5, st[1] >> 5

    def stats(pA, pB):
        total = lane_sum(pA + pB)[0]
        maxl = lane_max(jnp.maximum(pA, pB))[0]
        return total, maxl

    pA0, pB0 = extract(jnp.float32(T0))
    tot0, max0 = stats(pA0, pB0)

    def f2key(f):
        bits = lax.bitcast_convert_type(f, jnp.uint32)
        return jnp.where((bits >> jnp.uint32(31)) == jnp.uint32(0),
                         bits ^ jnp.uint32(0x80000000), ~bits)

    def key2f(kk):
        bits = jnp.where(kk >= jnp.uint32(0x80000000),
                         kk ^ jnp.uint32(0x80000000), ~kk)
        return lax.bitcast_convert_type(bits, jnp.float32)

    def invalid(total, maxl):
        return (total < K) | (total > CAPT) | (maxl > DEPTH)

    t0k = f2key(jnp.float32(T0))
    too_many0 = (tot0 > CAPT) | (max0 > DEPTH)
    lo0 = jnp.where(too_many0, t0k, jnp.uint32(0))
    hi0 = jnp.where(tot0 < K, t0k, jnp.uint32(0xFFFFFFFF))

    def cond(st):
        _, _, _, _, total, maxl, it = st
        return invalid(total, maxl) & (it < jnp.int32(40))

    def bod(st):
        lo, hi, _, _, _, _, it = st
        mid = lo + (hi - lo) // jnp.uint32(2)
        pA, pB = extract(key2f(mid))
        total, maxl = stats(pA, pB)
        too_many = (total > CAPT) | (maxl > DEPTH)
        lo2 = jnp.where(too_many, mid, lo)
        hi2 = jnp.where(total < K, mid, hi)
        return (lo2, hi2, pA, pB, total, maxl, it + 1)

    _, _, pA, pB, total, maxl, _ = lax.while_loop(
        cond, bod, (lo0, hi0, pA0, pB0, tot0, max0, jnp.int32(0)))

    neg_v = jnp.full((16,), NEG, jnp.float32)
    imax_v = jnp.full((16,), IMAX, jnp.int32)

    def cfill(i, _):
        ccv[pl.ds(i * 16, 16)] = neg_v
        cci[pl.ds(i * 16, 16)] = imax_v
        return 0

    lax.fori_loop(0, CAPT // 16, cfill, 0)

    totA = lane_sum(pA)
    base_a = lane_exclusive_prefix(pA)
    base_b = lane_exclusive_prefix(pB) + totA
    cap_v = jnp.full((16,), CAPT - 1, jnp.int32)
    maxd = jnp.minimum(maxl, jnp.int32(DEPTH))

    def cbody(d, _):
        vals = cv[pl.ds(d * 32, 16)]
        idxs = ci[pl.ds(d * 32, 16)]
        mk2 = pA > d
        tgt = jnp.minimum(base_a + d, cap_v)
        plsc.store_scatter(ccv, [tgt], vals, mask=mk2)
        plsc.store_scatter(cci, [tgt], idxs, mask=mk2)
        valsb = cv[pl.ds(d * 32 + 16, 16)]
        idxsb = ci[pl.ds(d * 32 + 16, 16)]
        mk3 = pB > d
        tgtb = jnp.minimum(base_b + d, cap_v)
        plsc.store_scatter(ccv, [tgtb], valsb, mask=mk3)
        plsc.store_scatter(cci, [tgtb], idxsb, mask=mk3)
        return 0

    lax.fori_loop(0, maxd, cbody, 0)

    cnt = jnp.minimum(total, jnp.int32(CAPT))
    nb = (cnt + 15) // 16

    rots = [(iota + r) % 16 for r in range(16)]

    def rbody(bi, _):
        b16 = bi * 16
        vt = ccv[pl.ds(b16, 16)]
        it_ = cci[pl.ds(b16, 16)]

        def inner(bs, acc):
            s16 = bs * 16
            for r in range(16):
                idxv = s16 + rots[r]
                vsr = plsc.load_gather(ccv, [idxv])
                isr = plsc.load_gather(cci, [idxv])
                c = (vsr > vt) | ((vsr == vt) & (isr < it_))
                acc = acc + jnp.where(c, one_v, zero_v)
            return acc

        acc = lax.fori_loop(0, nb, inner, zero_v)
        cr[pl.ds(b16, 16)] = acc
        return 0

    lax.fori_loop(0, nb, rbody, 0)

    kv = jnp.full((16,), K, jnp.int32)
    swv = sclv[pl.ds(0, 16)]
    shv = sclv[pl.ds(16, 16)]
    rcpC = jnp.float32(1.0 / NUM_CLASSES)

    def obody(b, _):
        vb = ccv[pl.ds(b * 16, 16)]
        ib = cci[pl.ds(b * 16, 16)]
        rb = cr[pl.ds(b * 16, 16)]
        msk = rb < kv
        rbc = jnp.where(msk, rb, kv)
        s = 1.0 / (1.0 + jnp.exp(-vb))
        q = ((ib.astype(jnp.float32) + 0.5) * rcpC).astype(jnp.int32)
        labv = ib - q * NUM_CLASSES
        plsc.store_scatter(osco, [rbc], s, mask=msk)
        plsc.store_scatter(olab, [rbc], labv, mask=msk)
        q_ = jnp.where(msk, q, zero_v)
        cx = plsc.load_gather(boxr, [q_])
        cy = plsc.load_gather(boxr, [q_ + Q])
        wv = plsc.load_gather(boxr, [q_ + 2 * Q])
        hv = plsc.load_gather(boxr, [q_ + 3 * Q])
        plsc.store_scatter(ox0, [rbc], (cx - 0.5 * wv) * swv, mask=msk)
        plsc.store_scatter(oy0, [rbc], (cy - 0.5 * hv) * shv, mask=msk)
        plsc.store_scatter(ox1, [rbc], (cx + 0.5 * wv) * swv, mask=msk)
        plsc.store_scatter(oy1, [rbc], (cy + 0.5 * hv) * shv, mask=msk)
        return 0

    lax.fori_loop(0, nb, obody, 0)

    pltpu.sync_copy(olab, lab_o.at[pl.ds(row * OUTP, OUTP)])
    pltpu.sync_copy(osco, sco_o.at[pl.ds(row * OUTP, OUTP)])
    pltpu.sync_copy(ox0, x0_o.at[pl.ds(row * OUTP, OUTP)])
    pltpu.sync_copy(oy0, y0_o.at[pl.ds(row * OUTP, OUTP)])
    pltpu.sync_copy(ox1, x1_o.at[pl.ds(row * OUTP, OUTP)])
    pltpu.sync_copy(oy1, y1_o.at[pl.ds(row * OUTP, OUTP)])


_mesh = plsc.VectorSubcoreMesh(core_axis_name="c", subcore_axis_name="s",
                               num_cores=2, num_subcores=16)

_f32 = jnp.float32
_i32 = jnp.int32

_sc_call = pl.kernel(
    _sc_body,
    out_type=(
        jax.ShapeDtypeStruct((B * OUTP,), _i32),
        jax.ShapeDtypeStruct((B * OUTP,), _f32),
        jax.ShapeDtypeStruct((B * OUTP,), _f32),
        jax.ShapeDtypeStruct((B * OUTP,), _f32),
        jax.ShapeDtypeStruct((B * OUTP,), _f32),
        jax.ShapeDtypeStruct((B * OUTP,), _f32),
    ),
    mesh=_mesh,
    compiler_params=pltpu.CompilerParams(needs_layout_passes=False),
    scratch_types=[
        pltpu.VMEM((2 * W,), _f32),
        pltpu.VMEM((2 * CAPT,), _f32),
        pltpu.VMEM((2 * CAPT,), _i32),
        pltpu.VMEM((CAPT,), _f32),
        pltpu.VMEM((CAPT,), _i32),
        pltpu.VMEM((CAPT,), _i32),
        pltpu.VMEM((Q4,), _f32),
        pltpu.VMEM((32,), _f32),
        pltpu.VMEM((16,), _i32),
        pltpu.VMEM((OUTP,), _i32),
        pltpu.VMEM((OUTP,), _f32),
        pltpu.VMEM((OUTP,), _f32),
        pltpu.VMEM((OUTP,), _f32),
        pltpu.VMEM((OUTP,), _f32),
        pltpu.VMEM((OUTP,), _f32),
        pltpu.SemaphoreType.DMA,
    ],
)


def kernel(pred_logits, pred_boxes, model_input_sizes):
    logits = pred_logits.transpose(0, 2, 1).reshape(-1)
    boxesf = pred_boxes.transpose(0, 2, 1).reshape(-1)
    msf = model_input_sizes.astype(jnp.float32)
    scale32 = jnp.concatenate(
        [jnp.broadcast_to(msf[:, :1], (B, 16)),
         jnp.broadcast_to(msf[:, 1:2], (B, 16))], axis=1).reshape(-1)
    lab, sco, x0, y0, x1, y1 = _sc_call(logits, boxesf, scale32)
    lab = lab.reshape(B, OUTP)[:, :K]
    sco = sco.reshape(B, OUTP)[:, :K]
    boxes = jnp.stack(
        [x0.reshape(B, OUTP)[:, :K], y0.reshape(B, OUTP)[:, :K],
         x1.reshape(B, OUTP)[:, :K], y1.reshape(B, OUTP)[:, :K]], axis=-1)
    return lab, boxes, sco
